# baseline (speedup 1.0000x reference)
import functools

import jax
import jax.numpy as jnp
import numpy as np
from jax import lax
from jax.experimental import pallas as pl
from jax.experimental.pallas import tpu as pltpu
from jax.experimental.pallas import tpu_sc as plsc

_B, _T, _K, _D = 16, 1024, 1024, 64
_NTOK = _B * _T
_TT = 1024
_X_TC = 13 * 1024

_NC, _NS, _L = 2, 16, 16
_NW = _NC * _NS
_SC_T0 = _X_TC
_SC_TOK = _NTOK - _X_TC
_TPW = _SC_TOK // _NW
_GRP = _TPW // _L

_K0 = np.uint32(0)
_K1 = np.uint32(42)
_K2 = np.uint32(0 ^ 42 ^ 0x1BD11BDA)
_ROT = ((13, 15, 26, 6), (17, 29, 16, 24))
_TINY = np.float32(np.finfo(np.float32).tiny)
_LN2 = np.float32(0.6931471805599453)


def _threefry_bits(x1_keyed):
    ks = (_K0, _K1, _K2)
    x1 = x1_keyed
    x0 = x1
    x1 = ((x1 << np.uint32(13)) | (x1 >> np.uint32(19))) ^ x0
    for r in _ROT[0][1:]:
        x0 = x0 + x1
        x1 = (x1 << np.uint32(r)) | (x1 >> np.uint32(32 - r))
        x1 = x1 ^ x0
    x0 = x0 + ks[1]
    x1 = x1 + np.uint32(ks[2] + np.uint32(1))
    for i in range(1, 5):
        for r in _ROT[i % 2]:
            x0 = x0 + x1
            x1 = (x1 << np.uint32(r)) | (x1 >> np.uint32(32 - r))
            x1 = x1 ^ x0
        x0 = x0 + ks[(i + 1) % 3]
        x1 = x1 + np.uint32(ks[(i + 2) % 3] + np.uint32(i + 1))
    return x0 ^ x1


def _bits_to_u(bits):
    fb = (bits >> np.uint32(9)) | np.uint32(0x3F800000)
    u01 = jax.lax.bitcast_convert_type(fb, jnp.float32) - jnp.float32(1.0)
    return u01 + _TINY


def _tc_kernel(logits_ref, z_ref, cent_ref, out_ref):
    t = pl.program_id(0)

    row = jax.lax.broadcasted_iota(jnp.uint32, (_TT, _K), 0)
    col = jax.lax.broadcasted_iota(jnp.int32, (_TT, _K), 1)
    base = (t * (_TT * _K)).astype(jnp.uint32) + _K1
    x1 = (row << np.uint32(10)) + col.astype(jnp.uint32) + base

    u = _bits_to_u(_threefry_bits(x1))
    g = -jnp.log(-jnp.log(u))
    s = logits_ref[...] + g

    m = jnp.max(s, axis=1, keepdims=True)
    one_hot = (s == m).astype(jnp.float32)
    q = jnp.dot(one_hot, cent_ref[...], preferred_element_type=jnp.float32)
    diff = z_ref[...] - q
    part = jnp.sum(diff * diff)

    @pl.when(t == 0)
    def _():
        out_ref[0, 0] = jnp.float32(0.0)

    out_ref[0, 0] += part


def _tc_partial(z2, logits2, centroids):
    out = pl.pallas_call(
        _tc_kernel,
        grid=(_X_TC // _TT,),
        in_specs=[
            pl.BlockSpec((_TT, _K), lambda t: (t, 0)),
            pl.BlockSpec((_TT, _D), lambda t: (t, 0)),
            pl.BlockSpec((_K, _D), lambda t: (0, 0)),
        ],
        out_specs=pl.BlockSpec(memory_space=pltpu.SMEM),
        out_shape=jax.ShapeDtypeStruct((1, 1), jnp.float32),
    )(logits2, z2, centroids)
    return out[0, 0]


def _log_sc(x):
    ix = jax.lax.bitcast_convert_type(x, jnp.int32)
    e = (ix >> 23) - 127
    mb = (ix & np.int32(0x007FFFFF)) | np.int32(0x3F800000)
    big = mb >= np.int32(0x3FB504F3)
    mb = jnp.where(big, mb - np.int32(0x00800000), mb)
    e = jnp.where(big, e + 1, e)
    m = jax.lax.bitcast_convert_type(mb, jnp.float32)
    s = (m - 1.0) / (m + 1.0)
    w = s * s
    p = np.float32(2.0 / 9.0)
    p = p * w + np.float32(2.0 / 7.0)
    p = p * w + np.float32(2.0 / 5.0)
    p = p * w + np.float32(2.0 / 3.0)
    p = p * w + np.float32(2.0)
    return e.astype(jnp.float32) * _LN2 + s * p


def _sc_loss(logits2, z2, centroids):
    mesh = plsc.VectorSubcoreMesh(core_axis_name="c", subcore_axis_name="s")

    @functools.partial(
        pl.kernel,
        mesh=mesh,
        out_type=jax.ShapeDtypeStruct((_NW, _L), jnp.float32),
        scratch_types=[
            pltpu.VMEM((_L, _K), jnp.float32),
            pltpu.VMEM((_L, _D), jnp.float32),
            pltpu.VMEM((_K // 2, 2 * _D), jnp.float32),
            pltpu.VMEM((_L,), jnp.float32),
            pltpu.SemaphoreType.DMA,
        ],
        compiler_params=pltpu.CompilerParams(needs_layout_passes=False),
    )
    def k(logits_hbm, z_hbm, cent_hbm, out_hbm, lg_v, z_v, cent_v, out_v, sem):
        wid = lax.axis_index("s") * _NC + lax.axis_index("c")
        lane = jax.lax.iota(jnp.int32, _L)
        lane_off = lane.astype(jnp.uint32) * np.uint32(_K)
        pos_inf = jnp.full((_L,), np.inf, dtype=jnp.float32)
        zeros_i = jnp.zeros((_L,), jnp.int32)

        pltpu.sync_copy(cent_hbm, cent_v)

        def gbody(grp, acc):
            t0 = _SC_T0 + wid * _TPW + grp * _L
            pltpu.sync_copy(logits_hbm.at[pl.ds(t0, _L), :], lg_v)
            pltpu.sync_copy(z_hbm.at[pl.ds(t0, _L), :], z_v)

            kbase = t0.astype(jnp.uint32) * np.uint32(_K) + _K1

            def body(kk, carry):
                best, bestk = carry
                ksp = jnp.full((_L,), kk, jnp.int32)
                x1 = lane_off + (kbase + kk.astype(jnp.uint32))
                u = _bits_to_u(_threefry_bits(x1))
                e = -_log_sc(u)
                lg = plsc.load_gather(lg_v, [lane, ksp])
                sp = e * jnp.exp(-lg)
                upd = sp < best
                best = jnp.where(upd, sp, best)
                bestk = jnp.where(upd, ksp, bestk)
                return best, bestk

            best, bestk = lax.fori_loop(0, _K, body, (pos_inf, zeros_i),
                                        unroll=4)

            crow = bestk >> 1
            cbase = (bestk & 1) << 6

            def dbody(d, a):
                dcol = jnp.full((_L,), d, jnp.int32)
                zc = plsc.load_gather(z_v, [lane, dcol])
                cc = plsc.load_gather(cent_v, [crow, cbase + dcol])
                dz = zc - cc
                return a + dz * dz

            return lax.fori_loop(0, _D, dbody, acc)

        acc = lax.fori_loop(0, _GRP, gbody, jnp.zeros((_L,), jnp.float32))
        out_v[...] = acc
        pltpu.sync_copy(out_v, out_hbm.at[wid])

    return k(logits2, z2, centroids.reshape(_K // 2, 2 * _D))


@jax.jit
def _run(z, cluster_logits, centroids):
    logits2 = cluster_logits.reshape(_NTOK, _K)
    z2 = z.reshape(_NTOK, _D)
    sc_parts = _sc_loss(logits2, z2, centroids)
    tc_part = _tc_partial(z2, logits2, centroids)
    return (tc_part + jnp.sum(sc_parts)) / jnp.float32(_B * _T * _D)


def kernel(z, cluster_logits, temperature, centroids):
    del temperature
    return _run(z, cluster_logits, centroids)


# speedup vs baseline: 1.4497x; 1.4497x over previous
import functools

import jax
import jax.numpy as jnp
import numpy as np
from jax import lax
from jax.experimental import pallas as pl
from jax.experimental.pallas import tpu as pltpu
from jax.experimental.pallas import tpu_sc as plsc

_B, _T, _K, _D = 16, 1024, 1024, 64
_NTOK = _B * _T
_TT = 1024
_X_TC = 13 * 1024

_NC, _NS, _L = 2, 16, 16
_NW = _NC * _NS
_SC_T0 = _X_TC
_SC_TOK = _NTOK - _X_TC
_TPW = _SC_TOK // _NW
_GRP = _TPW // _L

_K0 = np.uint32(0)
_K1 = np.uint32(42)
_K2 = np.uint32(0 ^ 42 ^ 0x1BD11BDA)
_ROT = ((13, 15, 26, 6), (17, 29, 16, 24))
_TINY = np.float32(np.finfo(np.float32).tiny)
_LN2 = np.float32(0.6931471805599453)


def _threefry_bits(x1_keyed):
    ks = (_K0, _K1, _K2)
    x1 = x1_keyed
    x0 = x1
    x1 = ((x1 << np.uint32(13)) | (x1 >> np.uint32(19))) ^ x0
    for r in _ROT[0][1:]:
        x0 = x0 + x1
        x1 = (x1 << np.uint32(r)) | (x1 >> np.uint32(32 - r))
        x1 = x1 ^ x0
    x0 = x0 + ks[1]
    x1 = x1 + np.uint32(ks[2] + np.uint32(1))
    for i in range(1, 5):
        for r in _ROT[i % 2]:
            x0 = x0 + x1
            x1 = (x1 << np.uint32(r)) | (x1 >> np.uint32(32 - r))
            x1 = x1 ^ x0
        x0 = x0 + ks[(i + 1) % 3]
        x1 = x1 + np.uint32(ks[(i + 2) % 3] + np.uint32(i + 1))
    return x0 ^ x1


def _bits_to_u(bits):
    fb = (bits >> np.uint32(9)) | np.uint32(0x3F800000)
    u01 = jax.lax.bitcast_convert_type(fb, jnp.float32) - jnp.float32(1.0)
    return u01 + _TINY


def _tc_kernel(logits_ref, z_ref, cent_ref, out_ref):
    t = pl.program_id(0)

    row = jax.lax.broadcasted_iota(jnp.uint32, (_TT, _K), 0)
    col = jax.lax.broadcasted_iota(jnp.int32, (_TT, _K), 1)
    base = (t * (_TT * _K)).astype(jnp.uint32) + _K1
    x1 = (row << np.uint32(10)) + col.astype(jnp.uint32) + base

    u = _bits_to_u(_threefry_bits(x1))
    g = -jnp.log(-jnp.log(u))
    s = logits_ref[...] + g

    m = jnp.max(s, axis=1, keepdims=True)
    idx = jnp.min(jnp.where(s == m, col, _K), axis=1)

    one_hot = (col == idx[:, None]).astype(jnp.float32)
    q = jnp.dot(one_hot, cent_ref[...], preferred_element_type=jnp.float32)
    diff = z_ref[...] - q
    part = jnp.sum(diff * diff)

    @pl.when(t == 0)
    def _():
        out_ref[0, 0] = jnp.float32(0.0)

    out_ref[0, 0] += part


def _tc_partial(z2, logits2, centroids):
    out = pl.pallas_call(
        _tc_kernel,
        grid=(_X_TC // _TT,),
        in_specs=[
            pl.BlockSpec((_TT, _K), lambda t: (t, 0)),
            pl.BlockSpec((_TT, _D), lambda t: (t, 0)),
            pl.BlockSpec((_K, _D), lambda t: (0, 0)),
        ],
        out_specs=pl.BlockSpec(memory_space=pltpu.SMEM),
        out_shape=jax.ShapeDtypeStruct((1, 1), jnp.float32),
    )(logits2, z2, centroids)
    return out[0, 0]


def _log_sc(x):
    ix = jax.lax.bitcast_convert_type(x, jnp.int32)
    e = (ix >> 23) - 127
    mb = (ix & np.int32(0x007FFFFF)) | np.int32(0x3F800000)
    big = mb >= np.int32(0x3FB504F3)
    mb = jnp.where(big, mb - np.int32(0x00800000), mb)
    e = jnp.where(big, e + 1, e)
    m = jax.lax.bitcast_convert_type(mb, jnp.float32)
    s = (m - 1.0) / (m + 1.0)
    w = s * s
    p = np.float32(2.0 / 9.0)
    p = p * w + np.float32(2.0 / 7.0)
    p = p * w + np.float32(2.0 / 5.0)
    p = p * w + np.float32(2.0 / 3.0)
    p = p * w + np.float32(2.0)
    return e.astype(jnp.float32) * _LN2 + s * p


def _sc_loss(logits2, z2, centroids):
    mesh = plsc.VectorSubcoreMesh(core_axis_name="c", subcore_axis_name="s")

    @functools.partial(
        pl.kernel,
        mesh=mesh,
        out_type=jax.ShapeDtypeStruct((_NW, _L), jnp.float32),
        scratch_types=[
            pltpu.VMEM((_L, _K), jnp.float32),
            pltpu.VMEM((_L, _D), jnp.float32),
            pltpu.VMEM((_K // 2, 2 * _D), jnp.float32),
            pltpu.VMEM((_L,), jnp.float32),
            pltpu.SemaphoreType.DMA,
        ],
        compiler_params=pltpu.CompilerParams(needs_layout_passes=False),
    )
    def k(logits_hbm, z_hbm, cent_hbm, out_hbm, lg_v, z_v, cent_v, out_v, sem):
        wid = lax.axis_index("s") * _NC + lax.axis_index("c")
        lane = jax.lax.iota(jnp.int32, _L)
        lane_off = lane.astype(jnp.uint32) * np.uint32(_K)
        pos_inf = jnp.full((_L,), np.inf, dtype=jnp.float32)
        zeros_i = jnp.zeros((_L,), jnp.int32)

        pltpu.sync_copy(cent_hbm, cent_v)

        def gbody(grp, acc):
            t0 = _SC_T0 + wid * _TPW + grp * _L
            pltpu.sync_copy(logits_hbm.at[pl.ds(t0, _L), :], lg_v)
            pltpu.sync_copy(z_hbm.at[pl.ds(t0, _L), :], z_v)

            kbase = t0.astype(jnp.uint32) * np.uint32(_K) + _K1

            def body(kk, carry):
                best, bestk = carry
                ksp = jnp.full((_L,), kk, jnp.int32)
                x1 = lane_off + (kbase + kk.astype(jnp.uint32))
                u = _bits_to_u(_threefry_bits(x1))
                e = -_log_sc(u)
                lg = plsc.load_gather(lg_v, [lane, ksp])
                sp = e * jnp.exp(-lg)
                upd = sp < best
                best = jnp.where(upd, sp, best)
                bestk = jnp.where(upd, ksp, bestk)
                return best, bestk

            best, bestk = lax.fori_loop(0, _K, body, (pos_inf, zeros_i),
                                        unroll=4)

            crow = bestk >> 1
            cbase = (bestk & 1) << 6

            def dbody(d, a):
                dcol = jnp.full((_L,), d, jnp.int32)
                zc = plsc.load_gather(z_v, [lane, dcol])
                cc = plsc.load_gather(cent_v, [crow, cbase + dcol])
                dz = zc - cc
                return a + dz * dz

            return lax.fori_loop(0, _D, dbody, acc)

        acc = lax.fori_loop(0, _GRP, gbody, jnp.zeros((_L,), jnp.float32))
        out_v[...] = acc
        pltpu.sync_copy(out_v, out_hbm.at[wid])

    return k(logits2, z2, centroids.reshape(_K // 2, 2 * _D))


@jax.jit
def _run(z, cluster_logits, centroids):
    logits2 = cluster_logits.reshape(_NTOK, _K)
    z2 = z.reshape(_NTOK, _D)
    sc_parts = _sc_loss(logits2, z2, centroids)
    tc_part = _tc_partial(z2, logits2, centroids)
    return (tc_part + jnp.sum(sc_parts)) / jnp.float32(_B * _T * _D)


def kernel(z, cluster_logits, temperature, centroids):
    del temperature
    return _run(z, cluster_logits, centroids)
